# Initial kernel scaffold; baseline (speedup 1.0000x reference)
import jax
import jax.numpy as jnp
from jax.experimental import pallas as pl


def kernel(hour_weekday, emb_hour, emb_weekday):
    raise NotImplementedError("write your pallas kernel here")


# speedup vs baseline: 1.8022x; 1.8022x over previous
import functools

import jax
import jax.numpy as jnp
from jax import lax
from jax.experimental import pallas as pl
from jax.experimental.pallas import tpu as pltpu
from jax.experimental.pallas import tpu_sc as plsc

B = 16384
D = 32
NC, NS, L = 2, 16, 16
NW = NC * NS
PER_W = B // NW
NCH = 4
CH = PER_W // NCH
HR = 24
TR = HR * 8
TW = 2 * D


def _table_body(h_ref, w_ref, out_ref):
    rows = lax.broadcasted_iota(jnp.int32, (TR, HR), 0)
    cols = lax.broadcasted_iota(jnp.int32, (TR, HR), 1)
    oh_h = ((rows >> 3) == cols).astype(jnp.float32)
    oh_w = ((rows[:, :7] & 7) == cols[:, :7]).astype(jnp.float32)
    hpart = jnp.dot(oh_h, h_ref[...], preferred_element_type=jnp.float32,
                    precision=lax.Precision.HIGHEST)
    wpart = jnp.dot(oh_w, w_ref[...], preferred_element_type=jnp.float32,
                    precision=lax.Precision.HIGHEST)
    out_ref[...] = jnp.concatenate([hpart, wpart], axis=1)


_build_table = pl.pallas_call(
    _table_body,
    out_shape=jax.ShapeDtypeStruct((TR, TW), jnp.float32),
)


def _gather_body(table_hbm, hw_hbm, out_hbm, hw_v, idx_v, rows_v, sem):
    wid = lax.axis_index("s") * NC + lax.axis_index("c")
    base = wid * PER_W
    pltpu.sync_copy(hw_hbm.at[pl.ds(base, PER_W), :], hw_v)
    lane = lax.iota(jnp.int32, L)
    czero = lane * 0
    cone = czero + 1

    def fuse(j, _):
        ridx = j * L + lane
        h = plsc.load_gather(hw_v, [ridx, czero])
        w = plsc.load_gather(hw_v, [ridx, cone])
        idx_v[pl.ds(j * L, L)] = (h << 3) + w
        return 0

    lax.fori_loop(0, PER_W // L, fuse, 0)
    copies = [
        pltpu.async_copy(table_hbm.at[idx_v.at[pl.ds(k * CH, CH)]],
                         rows_v.at[k], sem)
        for k in range(NCH)
    ]
    for k in range(NCH):
        copies[k].wait()
        pltpu.sync_copy(rows_v.at[k], out_hbm.at[pl.ds(base + k * CH, CH), :])


_sc_gather = functools.partial(
    pl.kernel,
    out_type=jax.ShapeDtypeStruct((B, 2 * D), jnp.float32),
    mesh=plsc.VectorSubcoreMesh(core_axis_name="c", subcore_axis_name="s"),
    scratch_types=[
        pltpu.VMEM((PER_W, 2), jnp.int32),
        pltpu.VMEM((PER_W,), jnp.int32),
        pltpu.VMEM((NCH, CH, TW), jnp.float32),
        pltpu.SemaphoreType.DMA,
    ],
    compiler_params=pltpu.CompilerParams(
        needs_layout_passes=False, use_tc_tiling_on_sc=False),
)(_gather_body)


def kernel(hour_weekday, emb_hour, emb_weekday):
    table = _build_table(emb_hour, emb_weekday)
    return _sc_gather(table, hour_weekday.astype(jnp.int32))
